# Initial kernel scaffold; baseline (speedup 1.0000x reference)
import jax
import jax.numpy as jnp
from jax.experimental import pallas as pl


def kernel(x, edge_index, iteration, tok_emb, prev_emb, iter_emb, W_pos, b_pos, W_in, b_in, W_enc0, b_enc0, W_bn0, b_bn0, W_bn1, b_bn1, W_dec0, b_dec0, W_pred, b_pred, W_conf, b_conf):
    raise NotImplementedError("write your pallas kernel here")


# speedup vs baseline: 8.0030x; 8.0030x over previous
import functools
import jax
import jax.numpy as jnp
from jax import lax
from jax.experimental import pallas as pl
from jax.experimental.pallas import tpu as pltpu
from jax.experimental.pallas import tpu_sc as plsc

_NC = 2
_NS = 16
_NW = _NC * _NS
_CH = 80


def _round_up(v, m):
    return (v + m - 1) // m * m


def _sc_mesh():
    return plsc.VectorSubcoreMesh(core_axis_name="c", subcore_axis_name="s")


def _zero_2d(ref, nrows, ncols):
    npc = ncols // 16

    def zr(i, _):
        r = i // npc
        col = (i % npc) * 16
        ref[r, pl.ds(col, 16)] = jnp.zeros((16,), jnp.float32)
        return 0

    lax.fori_loop(0, nrows * npc, zr, 0)


def _make_embed_kernel(NP, EPAD, NACC, DV, DP):
    n_chunks = NP // _CH // _NW
    e_per_w = EPAD // _NW
    e_chunks = e_per_w // _CH
    rows_per_tile = NACC // _NS
    n_slabs = rows_per_tile // _CH

    @functools.partial(
        pl.kernel,
        out_type=(
            jax.ShapeDtypeStruct((NP, DV), jnp.float32),
            jax.ShapeDtypeStruct((NP, DP), jnp.float32),
            jax.ShapeDtypeStruct((_NC, NACC, 128), jnp.float32),
        ),
        mesh=_sc_mesh(),
        scratch_types=[
            pltpu.VMEM((_CH,), jnp.int32),
            pltpu.VMEM((_CH, DV), jnp.float32),
            pltpu.VMEM((_CH, DP), jnp.float32),
            pltpu.VMEM((_CH, 128), jnp.float32),
            pltpu.VMEM((_CH, 128), jnp.float32),
            pltpu.VMEM_SHARED((NACC, 128), jnp.float32),
            pltpu.SemaphoreType.DMA,
        ],
    )
    def embed_kernel(tok_tab, prev_tab, tok_ids, prev_ids, dst_hbm,
                     tok_out, prev_out, deg_out,
                     idx, trows, prows, ones, zeros, dacc, sem):
        c = lax.axis_index("c")
        s = lax.axis_index("s")
        wid = s * _NC + c

        def fill(i, _):
            r = i // 8
            col = (i % 8) * 16
            ones[r, pl.ds(col, 16)] = jnp.ones((16,), jnp.float32)
            zeros[r, pl.ds(col, 16)] = jnp.zeros((16,), jnp.float32)
            return 0

        lax.fori_loop(0, _CH * 8, fill, 0)

        base_r = s * rows_per_tile
        for z in range(n_slabs):
            pltpu.sync_copy(zeros, dacc.at[pl.ds(base_r + z * _CH, _CH)])
        plsc.subcore_barrier()

        def emb_step(t, _):
            base = (wid * n_chunks + t) * _CH
            pltpu.sync_copy(tok_ids.at[pl.ds(base, _CH)], idx)
            pltpu.async_copy(tok_tab.at[idx], trows, sem).wait()
            pltpu.sync_copy(trows, tok_out.at[pl.ds(base, _CH)])
            pltpu.sync_copy(prev_ids.at[pl.ds(base, _CH)], idx)
            pltpu.async_copy(prev_tab.at[idx], prows, sem).wait()
            pltpu.sync_copy(prows, prev_out.at[pl.ds(base, _CH)])
            return 0

        lax.fori_loop(0, n_chunks, emb_step, 0)

        def deg_step(j, _):
            base = wid * e_per_w + j * _CH
            pltpu.sync_copy(dst_hbm.at[pl.ds(base, _CH)], idx)
            pltpu.sync_copy(ones, dacc.at[idx], add=True)
            return 0

        lax.fori_loop(0, e_chunks, deg_step, 0)
        plsc.subcore_barrier()

        for z in range(n_slabs):
            r0 = base_r + z * _CH
            pltpu.sync_copy(dacc.at[pl.ds(r0, _CH)],
                            deg_out.at[c, pl.ds(r0, _CH)])

    return embed_kernel


_NB = 2


def _make_scatter_kernel(NACC, EPAD, H):
    e_per_w = EPAD // _NW
    e_chunks = e_per_w // _CH
    assert e_chunks % _NB == 0
    rows_per_tile = NACC // _NS
    n_slabs = rows_per_tile // _CH

    @functools.partial(
        pl.kernel,
        out_type=jax.ShapeDtypeStruct((_NC, NACC, H), jnp.float32),
        mesh=_sc_mesh(),
        scratch_types=[
            [pltpu.VMEM((_CH,), jnp.int32) for _ in range(_NB)],
            [pltpu.VMEM((_CH,), jnp.int32) for _ in range(_NB)],
            [pltpu.VMEM((_CH, H), jnp.float32) for _ in range(_NB)],
            pltpu.VMEM((_CH, H), jnp.float32),
            pltpu.VMEM_SHARED((NACC, H), jnp.float32),
            [pltpu.SemaphoreType.DMA for _ in range(_NB)],
            [pltpu.SemaphoreType.DMA for _ in range(_NB)],
            [pltpu.SemaphoreType.DMA for _ in range(_NB)],
            [pltpu.SemaphoreType.DMA for _ in range(_NB)],
        ],
    )
    def scatter_kernel(g_hbm, src_hbm, dst_hbm, out_hbm,
                       sidx, didx, rows, zbuf, acc, isem, idsem, gsem, ssem):
        c = lax.axis_index("c")
        s = lax.axis_index("s")
        wid = s * _NC + c
        e0 = wid * e_per_w

        _zero_2d(zbuf, _CH, H)
        base_r = s * rows_per_tile
        for z in range(n_slabs):
            pltpu.sync_copy(zbuf, acc.at[pl.ds(base_r + z * _CH, _CH)])
        plsc.subcore_barrier()

        def group(t, _):
            j0 = e0 + t * _NB * _CH
            idescs = []
            for b in range(_NB):
                idescs.append(pltpu.async_copy(
                    src_hbm.at[pl.ds(j0 + b * _CH, _CH)], sidx[b], isem[b]))
                idescs.append(pltpu.async_copy(
                    dst_hbm.at[pl.ds(j0 + b * _CH, _CH)], didx[b], idsem[b]))
            descs = []
            for b in range(_NB):
                idescs[2 * b].wait()
                idescs[2 * b + 1].wait()
                descs.append(pltpu.async_copy(
                    g_hbm.at[sidx[b]], rows[b], gsem[b]))
            sdescs = []
            for b in range(_NB):
                descs[b].wait()
                sdescs.append(pltpu.async_copy(
                    rows[b], acc.at[didx[b]], ssem[b], add=True))
            for b in range(_NB):
                sdescs[b].wait()
            return 0

        lax.fori_loop(0, e_chunks // _NB, group, 0)
        plsc.subcore_barrier()

        for z in range(n_slabs):
            r0 = base_r + z * _CH
            pltpu.sync_copy(acc.at[pl.ds(r0, _CH)],
                            out_hbm.at[c, pl.ds(r0, _CH)])

    return scatter_kernel


def _gelu(x):
    return 0.5 * x * (1.0 + lax.erf(x * 0.7071067811865476))


def _tc_input_body(it_ref, tok_ref, prev_ref, pos_ref, degp_ref, iter_ref,
                   wpos_ref, bpos_ref, wa_ref, wb_ref, wc_ref, wd_ref,
                   bin_ref, wenc_ref, g_ref, dinv_ref):
    it = it_ref[0]
    B = tok_ref.shape[0]
    p = jnp.dot(pos_ref[...], wpos_ref[...],
                preferred_element_type=jnp.float32) + bpos_ref[...]
    itrow = jnp.broadcast_to(iter_ref[pl.ds(it, 1), :], (B, iter_ref.shape[1]))
    h = (jnp.dot(tok_ref[...], wa_ref[...], preferred_element_type=jnp.float32)
         + jnp.dot(prev_ref[...], wb_ref[...], preferred_element_type=jnp.float32)
         + jnp.dot(p, wc_ref[...], preferred_element_type=jnp.float32)
         + jnp.dot(itrow, wd_ref[...], preferred_element_type=jnp.float32)
         + bin_ref[...])
    h = _gelu(h)
    deg = degp_ref[0, :, 0:1] + degp_ref[1, :, 0:1] + 1.0
    dinv = lax.rsqrt(deg)
    dinv_ref[...] = dinv
    g_ref[...] = jnp.dot(h, wenc_ref[...],
                         preferred_element_type=jnp.float32) * dinv


def _tc_mid_body(acc_ref, g_ref, dinv_ref, b_ref, w_ref, gout_ref):
    dinv = dinv_ref[...]
    t = _gelu(dinv * (acc_ref[0] + acc_ref[1] + g_ref[...]) + b_ref[...])
    gout_ref[...] = jnp.dot(t, w_ref[...],
                            preferred_element_type=jnp.float32) * dinv


def _tc_head_body(acc_ref, g_ref, dinv_ref, b_ref, wpred_ref, bpred_ref,
                  wconf_ref, bconf_ref, logits_ref, conf_ref):
    dinv = dinv_ref[...]
    h = _gelu(dinv * (acc_ref[0] + acc_ref[1] + g_ref[...]) + b_ref[...])
    logits_ref[...] = jnp.dot(h, wpred_ref[...],
                              preferred_element_type=jnp.float32) + bpred_ref[...]
    conf_ref[...] = jax.nn.sigmoid(
        jnp.dot(h, wconf_ref[...], preferred_element_type=jnp.float32)
        + bconf_ref[...])


def kernel(x, edge_index, iteration, tok_emb, prev_emb, iter_emb, W_pos,
           b_pos, W_in, b_in, W_enc0, b_enc0, W_bn0, b_bn0, W_bn1, b_bn1,
           W_dec0, b_dec0, W_pred, b_pred, W_conf, b_conf):
    N = x.shape[0]
    E = edge_index.shape[1]
    H = W_in.shape[1]
    DV = tok_emb.shape[1]
    DP = prev_emb.shape[1]
    V = W_pred.shape[1]

    NP = _round_up(N + 1, _CH * _NW)
    EPAD = _round_up(E, _CH * _NW * _NB)

    DPP = _round_up(DP, 128)

    tok_ids = jnp.pad(x[:, 0], (0, NP - N))
    prev_ids = jnp.pad(x[:, 1], (0, NP - N))
    pos = jnp.pad(x[:, 2:4].astype(jnp.float32), ((0, NP - N), (0, 0)))
    src_p = jnp.pad(edge_index[0], (0, EPAD - E), constant_values=N)
    dst_p = jnp.pad(edge_index[1], (0, EPAD - E), constant_values=N)
    it_arr = jnp.asarray([iteration], dtype=jnp.int32)
    prev_emb_p = jnp.pad(prev_emb, ((0, 0), (0, DPP - DP)))

    Wa = W_in[:DV]
    Wb = jnp.pad(W_in[DV:DV + DP], ((0, DPP - DP), (0, 0)))
    Wc = W_in[DV + DP:DV + 2 * DP]
    Wd = W_in[DV + 2 * DP:]

    tok_rows, prev_rows, degp = _make_embed_kernel(NP, EPAD, NP, DV, DPP)(
        tok_emb, prev_emb_p, tok_ids, prev_ids, dst_p)

    B = 1024
    grid = NP // B
    row_spec = lambda w: pl.BlockSpec((B, w), lambda i: (i, 0))
    full_spec = lambda a, b: pl.BlockSpec((a, b), lambda i: (0, 0))
    acc_spec = pl.BlockSpec((_NC, B, H), lambda i: (0, i, 0))
    vec_spec = lambda w: pl.BlockSpec((w,), lambda i: (0,))

    g1, dinv = pl.pallas_call(
        _tc_input_body,
        grid=(grid,),
        in_specs=[
            pl.BlockSpec(memory_space=pltpu.SMEM),
            row_spec(DV), row_spec(DPP), row_spec(2),
            pl.BlockSpec((_NC, B, 128), lambda i: (0, i, 0)),
            full_spec(*iter_emb.shape),
            full_spec(2, DP), vec_spec(DP),
            full_spec(DV, H), full_spec(DPP, H), full_spec(DP, H),
            full_spec(DP, H), vec_spec(H), full_spec(H, H),
        ],
        out_specs=[row_spec(H), row_spec(1)],
        out_shape=[
            jax.ShapeDtypeStruct((NP, H), jnp.float32),
            jax.ShapeDtypeStruct((NP, 1), jnp.float32),
        ],
    )(it_arr, tok_rows, prev_rows, pos, degp, iter_emb,
      W_pos, b_pos, Wa, Wb, Wc, Wd, b_in, W_enc0)

    scatter = _make_scatter_kernel(NP, EPAD, H)

    mid = pl.pallas_call(
        _tc_mid_body,
        grid=(grid,),
        in_specs=[acc_spec, row_spec(H), row_spec(1), vec_spec(H),
                  full_spec(H, H)],
        out_specs=row_spec(H),
        out_shape=jax.ShapeDtypeStruct((NP, H), jnp.float32),
    )

    g = g1
    for b_cur, W_next in ((b_enc0, W_bn0), (b_bn0, W_bn1), (b_bn1, W_dec0)):
        acc = scatter(g, src_p, dst_p)
        g = mid(acc, g, dinv, b_cur, W_next)

    acc = scatter(g, src_p, dst_p)
    logits, conf = pl.pallas_call(
        _tc_head_body,
        grid=(grid,),
        in_specs=[acc_spec, row_spec(H), row_spec(1), vec_spec(H),
                  full_spec(H, V), vec_spec(V), full_spec(H, 1), vec_spec(1)],
        out_specs=[row_spec(V), row_spec(1)],
        out_shape=[
            jax.ShapeDtypeStruct((NP, V), jnp.float32),
            jax.ShapeDtypeStruct((NP, 1), jnp.float32),
        ],
    )(acc, g, dinv, b_dec0, W_pred, b_pred, W_conf, b_conf)

    return (logits[:N], conf[:N])


# speedup vs baseline: 8.3547x; 1.0440x over previous
import functools
import jax
import jax.numpy as jnp
from jax import lax
from jax.experimental import pallas as pl
from jax.experimental.pallas import tpu as pltpu
from jax.experimental.pallas import tpu_sc as plsc

_NC = 2
_NS = 16
_NW = _NC * _NS
_CH = 112


def _round_up(v, m):
    return (v + m - 1) // m * m


def _sc_mesh():
    return plsc.VectorSubcoreMesh(core_axis_name="c", subcore_axis_name="s")


def _fill_2d(ref, nrows, ncols, val):
    npc = ncols // 16

    def zr(i, _):
        r = i // npc
        col = (i % npc) * 16
        ref[r, pl.ds(col, 16)] = jnp.full((16,), val, jnp.float32)
        return 0

    lax.fori_loop(0, nrows * npc, zr, 0)


def _make_embed_kernel(NP, EPAD, NACC, DV, DP):
    n_chunks = NP // _CH // _NW
    e_per_w = EPAD // _NW
    e_chunks = e_per_w // _CH
    rows_per_tile = NACC // _NS
    n_slabs = rows_per_tile // _CH

    @functools.partial(
        pl.kernel,
        out_type=(
            jax.ShapeDtypeStruct((NP, DV), jnp.float32),
            jax.ShapeDtypeStruct((NP, DP), jnp.float32),
            jax.ShapeDtypeStruct((_NC, NACC, 128), jnp.float32),
        ),
        mesh=_sc_mesh(),
        scratch_types=[
            pltpu.VMEM((_CH,), jnp.int32),
            pltpu.VMEM((_CH, DV), jnp.float32),
            pltpu.VMEM((_CH, DP), jnp.float32),
            pltpu.VMEM_SHARED((NACC, 128), jnp.float32),
            pltpu.SemaphoreType.DMA,
        ],
    )
    def embed_kernel(tok_tab, prev_tab, tok_ids, prev_ids, dst_hbm,
                     tok_out, prev_out, deg_out,
                     idx, trows, prows, dacc, sem):
        c = lax.axis_index("c")
        s = lax.axis_index("s")
        wid = s * _NC + c

        _fill_2d(trows, _CH, DV, 0.0)

        base_r = s * rows_per_tile
        zdescs = [
            pltpu.async_copy(trows, dacc.at[pl.ds(base_r + z * _CH, _CH)],
                             sem)
            for z in range(n_slabs)]
        for d in zdescs:
            d.wait()
        plsc.subcore_barrier()

        def emb_step(t, _):
            base = (wid * n_chunks + t) * _CH
            pltpu.sync_copy(tok_ids.at[pl.ds(base, _CH)], idx)
            pltpu.async_copy(tok_tab.at[idx], trows, sem).wait()
            pltpu.sync_copy(trows, tok_out.at[pl.ds(base, _CH)])
            pltpu.sync_copy(prev_ids.at[pl.ds(base, _CH)], idx)
            pltpu.async_copy(prev_tab.at[idx], prows, sem).wait()
            pltpu.sync_copy(prows, prev_out.at[pl.ds(base, _CH)])
            return 0

        lax.fori_loop(0, n_chunks, emb_step, 0)

        _fill_2d(prows, _CH, 128, 1.0)

        def deg_step(j, _):
            base = wid * e_per_w + j * _CH
            pltpu.sync_copy(dst_hbm.at[pl.ds(base, _CH)], idx)
            pltpu.sync_copy(prows.at[:, pl.ds(0, 128)], dacc.at[idx], add=True)
            return 0

        lax.fori_loop(0, e_chunks, deg_step, 0)
        plsc.subcore_barrier()

        wdescs = [
            pltpu.async_copy(dacc.at[pl.ds(base_r + z * _CH, _CH)],
                             deg_out.at[c, pl.ds(base_r + z * _CH, _CH)],
                             sem)
            for z in range(n_slabs)]
        for d in wdescs:
            d.wait()

    return embed_kernel


_NB = 3


def _make_scatter_kernel(NACC, EPAD, H):
    e_per_w = EPAD // _NW
    e_chunks = e_per_w // _CH
    assert e_chunks % _NB == 0
    rows_per_tile = NACC // _NS
    n_slabs = rows_per_tile // _CH

    @functools.partial(
        pl.kernel,
        out_type=jax.ShapeDtypeStruct((_NC, NACC, H), jnp.float32),
        mesh=_sc_mesh(),
        scratch_types=[
            [pltpu.VMEM((_CH,), jnp.int32) for _ in range(_NB)],
            [pltpu.VMEM((_CH,), jnp.int32) for _ in range(_NB)],
            [pltpu.VMEM((_CH, H), jnp.float32) for _ in range(_NB)],
            pltpu.VMEM_SHARED((NACC, H), jnp.float32),
            [pltpu.SemaphoreType.DMA for _ in range(_NB)],
            [pltpu.SemaphoreType.DMA for _ in range(_NB)],
            [pltpu.SemaphoreType.DMA for _ in range(_NB)],
            [pltpu.SemaphoreType.DMA for _ in range(_NB)],
        ],
    )
    def scatter_kernel(g_hbm, src_hbm, dst_hbm, out_hbm,
                       sidx, didx, rows, acc, isem, idsem, gsem, ssem):
        c = lax.axis_index("c")
        s = lax.axis_index("s")
        wid = s * _NC + c
        e0 = wid * e_per_w

        _fill_2d(rows[0], _CH, H, 0.0)
        base_r = s * rows_per_tile
        zdescs = [
            pltpu.async_copy(rows[0], acc.at[pl.ds(base_r + z * _CH, _CH)],
                             gsem[0])
            for z in range(n_slabs)]
        for d in zdescs:
            d.wait()
        plsc.subcore_barrier()

        def group(t, _):
            j0 = e0 + t * _NB * _CH
            idescs = []
            for b in range(_NB):
                idescs.append(pltpu.async_copy(
                    src_hbm.at[pl.ds(j0 + b * _CH, _CH)], sidx[b], isem[b]))
                idescs.append(pltpu.async_copy(
                    dst_hbm.at[pl.ds(j0 + b * _CH, _CH)], didx[b], idsem[b]))
            descs = []
            for b in range(_NB):
                idescs[2 * b].wait()
                idescs[2 * b + 1].wait()
                descs.append(pltpu.async_copy(
                    g_hbm.at[sidx[b]], rows[b], gsem[b]))
            sdescs = []
            for b in range(_NB):
                descs[b].wait()
                sdescs.append(pltpu.async_copy(
                    rows[b], acc.at[didx[b]], ssem[b], add=True))
            for b in range(_NB):
                sdescs[b].wait()
            return 0

        lax.fori_loop(0, e_chunks // _NB, group, 0)
        plsc.subcore_barrier()

        wdescs = [
            pltpu.async_copy(acc.at[pl.ds(base_r + z * _CH, _CH)],
                             out_hbm.at[c, pl.ds(base_r + z * _CH, _CH)],
                             gsem[0])
            for z in range(n_slabs)]
        for d in wdescs:
            d.wait()

    return scatter_kernel


def _gelu(x):
    return 0.5 * x * (1.0 + lax.erf(x * 0.7071067811865476))


def _tc_input_body(it_ref, tok_ref, prev_ref, pos_ref, degp_ref, iter_ref,
                   wpos_ref, bpos_ref, wa_ref, wb_ref, wc_ref, wd_ref,
                   bin_ref, wenc_ref, g_ref, dinv_ref):
    it = it_ref[0]
    B = tok_ref.shape[0]
    p = jnp.dot(pos_ref[...], wpos_ref[...],
                preferred_element_type=jnp.float32) + bpos_ref[...]
    itrow = jnp.broadcast_to(iter_ref[pl.ds(it, 1), :], (B, iter_ref.shape[1]))
    h = (jnp.dot(tok_ref[...], wa_ref[...], preferred_element_type=jnp.float32)
         + jnp.dot(prev_ref[...], wb_ref[...], preferred_element_type=jnp.float32)
         + jnp.dot(p, wc_ref[...], preferred_element_type=jnp.float32)
         + jnp.dot(itrow, wd_ref[...], preferred_element_type=jnp.float32)
         + bin_ref[...])
    h = _gelu(h)
    deg = degp_ref[0, :, 0:1] + degp_ref[1, :, 0:1] + 1.0
    dinv = lax.rsqrt(deg)
    dinv_ref[...] = dinv
    g_ref[...] = jnp.dot(h, wenc_ref[...],
                         preferred_element_type=jnp.float32) * dinv


def _tc_mid_body(acc_ref, g_ref, dinv_ref, b_ref, w_ref, gout_ref):
    dinv = dinv_ref[...]
    t = _gelu(dinv * (acc_ref[0] + acc_ref[1] + g_ref[...]) + b_ref[...])
    gout_ref[...] = jnp.dot(t, w_ref[...],
                            preferred_element_type=jnp.float32) * dinv


def _tc_head_body(acc_ref, g_ref, dinv_ref, b_ref, wpred_ref, bpred_ref,
                  wconf_ref, bconf_ref, logits_ref, conf_ref):
    dinv = dinv_ref[...]
    h = _gelu(dinv * (acc_ref[0] + acc_ref[1] + g_ref[...]) + b_ref[...])
    logits_ref[...] = jnp.dot(h, wpred_ref[...],
                              preferred_element_type=jnp.float32) + bpred_ref[...]
    conf_ref[...] = jax.nn.sigmoid(
        jnp.dot(h, wconf_ref[...], preferred_element_type=jnp.float32)
        + bconf_ref[...])


def kernel(x, edge_index, iteration, tok_emb, prev_emb, iter_emb, W_pos,
           b_pos, W_in, b_in, W_enc0, b_enc0, W_bn0, b_bn0, W_bn1, b_bn1,
           W_dec0, b_dec0, W_pred, b_pred, W_conf, b_conf):
    N = x.shape[0]
    E = edge_index.shape[1]
    H = W_in.shape[1]
    DV = tok_emb.shape[1]
    DP = prev_emb.shape[1]
    V = W_pred.shape[1]

    NP = _round_up(N + 1, _CH * _NW)
    EPAD = _round_up(E, _CH * _NW * _NB)

    DPP = _round_up(DP, 128)

    tok_ids = jnp.pad(x[:, 0], (0, NP - N))
    prev_ids = jnp.pad(x[:, 1], (0, NP - N))
    pos = jnp.pad(x[:, 2:4].astype(jnp.float32), ((0, NP - N), (0, 0)))
    src_p = jnp.pad(edge_index[0], (0, EPAD - E), constant_values=N)
    dst_p = jnp.pad(edge_index[1], (0, EPAD - E), constant_values=N)
    it_arr = jnp.asarray([iteration], dtype=jnp.int32)
    prev_emb_p = jnp.pad(prev_emb, ((0, 0), (0, DPP - DP)))

    Wa = W_in[:DV]
    Wb = jnp.pad(W_in[DV:DV + DP], ((0, DPP - DP), (0, 0)))
    Wc = W_in[DV + DP:DV + 2 * DP]
    Wd = W_in[DV + 2 * DP:]

    tok_rows, prev_rows, degp = _make_embed_kernel(NP, EPAD, NP, DV, DPP)(
        tok_emb, prev_emb_p, tok_ids, prev_ids, dst_p)

    B = next(d for d in (1024, 896, 768, 640, 512, 448, 384, 256, 128)
             if NP % d == 0)
    grid = NP // B
    row_spec = lambda w: pl.BlockSpec((B, w), lambda i: (i, 0))
    full_spec = lambda a, b: pl.BlockSpec((a, b), lambda i: (0, 0))
    acc_spec = pl.BlockSpec((_NC, B, H), lambda i: (0, i, 0))
    vec_spec = lambda w: pl.BlockSpec((w,), lambda i: (0,))

    g1, dinv = pl.pallas_call(
        _tc_input_body,
        grid=(grid,),
        in_specs=[
            pl.BlockSpec(memory_space=pltpu.SMEM),
            row_spec(DV), row_spec(DPP), row_spec(2),
            pl.BlockSpec((_NC, B, 128), lambda i: (0, i, 0)),
            full_spec(*iter_emb.shape),
            full_spec(2, DP), vec_spec(DP),
            full_spec(DV, H), full_spec(DPP, H), full_spec(DP, H),
            full_spec(DP, H), vec_spec(H), full_spec(H, H),
        ],
        out_specs=[row_spec(H), row_spec(1)],
        out_shape=[
            jax.ShapeDtypeStruct((NP, H), jnp.float32),
            jax.ShapeDtypeStruct((NP, 1), jnp.float32),
        ],
    )(it_arr, tok_rows, prev_rows, pos, degp, iter_emb,
      W_pos, b_pos, Wa, Wb, Wc, Wd, b_in, W_enc0)

    scatter = _make_scatter_kernel(NP, EPAD, H)

    mid = pl.pallas_call(
        _tc_mid_body,
        grid=(grid,),
        in_specs=[acc_spec, row_spec(H), row_spec(1), vec_spec(H),
                  full_spec(H, H)],
        out_specs=row_spec(H),
        out_shape=jax.ShapeDtypeStruct((NP, H), jnp.float32),
    )

    g = g1
    for b_cur, W_next in ((b_enc0, W_bn0), (b_bn0, W_bn1), (b_bn1, W_dec0)):
        acc = scatter(g, src_p, dst_p)
        g = mid(acc, g, dinv, b_cur, W_next)

    acc = scatter(g, src_p, dst_p)
    logits, conf = pl.pallas_call(
        _tc_head_body,
        grid=(grid,),
        in_specs=[acc_spec, row_spec(H), row_spec(1), vec_spec(H),
                  full_spec(H, V), vec_spec(V), full_spec(H, 1), vec_spec(1)],
        out_specs=[row_spec(V), row_spec(1)],
        out_shape=[
            jax.ShapeDtypeStruct((NP, V), jnp.float32),
            jax.ShapeDtypeStruct((NP, 1), jnp.float32),
        ],
    )(acc, g, dinv, b_dec0, W_pred, b_pred, W_conf, b_conf)

    return (logits[:N], conf[:N])


# speedup vs baseline: 8.5041x; 1.0179x over previous
import functools
import jax
import jax.numpy as jnp
from jax import lax
from jax.experimental import pallas as pl
from jax.experimental.pallas import tpu as pltpu
from jax.experimental.pallas import tpu_sc as plsc

_NC = 2
_NS = 16
_NW = _NC * _NS
_CH = 112


def _round_up(v, m):
    return (v + m - 1) // m * m


def _sc_mesh():
    return plsc.VectorSubcoreMesh(core_axis_name="c", subcore_axis_name="s")


def _fill_2d(ref, nrows, ncols, val):
    npc = ncols // 16

    def zr(i, _):
        r = i // npc
        col = (i % npc) * 16
        ref[r, pl.ds(col, 16)] = jnp.full((16,), val, jnp.float32)
        return 0

    lax.fori_loop(0, nrows * npc, zr, 0)


def _make_embed_kernel(NP, EPAD, NACC, DV, DP):
    n_chunks = NP // _CH // _NW
    e_per_w = EPAD // _NW
    e_chunks = e_per_w // _CH
    rows_per_tile = NACC // _NS
    n_slabs = rows_per_tile // _CH

    @functools.partial(
        pl.kernel,
        out_type=(
            jax.ShapeDtypeStruct((NP, DV), jnp.float32),
            jax.ShapeDtypeStruct((NP, DP), jnp.float32),
            jax.ShapeDtypeStruct((_NC, NACC, 128), jnp.float32),
        ),
        mesh=_sc_mesh(),
        scratch_types=[
            pltpu.VMEM((_CH,), jnp.int32),
            pltpu.VMEM((_CH, DV), jnp.float32),
            pltpu.VMEM((_CH, DP), jnp.float32),
            [pltpu.VMEM((_CH,), jnp.int32) for _ in range(3)],
            pltpu.VMEM_SHARED((NACC, 128), jnp.float32),
            pltpu.SemaphoreType.DMA,
            [pltpu.SemaphoreType.DMA for _ in range(3)],
            [pltpu.SemaphoreType.DMA for _ in range(3)],
        ],
    )
    def embed_kernel(tok_tab, prev_tab, tok_ids, prev_ids, dst_hbm,
                     tok_out, prev_out, deg_out,
                     idx, trows, prows, didx, dacc, sem, isem, ssem):
        c = lax.axis_index("c")
        s = lax.axis_index("s")
        wid = s * _NC + c

        _fill_2d(trows, _CH, DV, 0.0)

        base_r = s * rows_per_tile
        zdescs = [
            pltpu.async_copy(trows, dacc.at[pl.ds(base_r + z * _CH, _CH)],
                             sem)
            for z in range(n_slabs)]
        for d in zdescs:
            d.wait()
        plsc.subcore_barrier()

        def emb_step(t, _):
            base = (wid * n_chunks + t) * _CH
            pltpu.sync_copy(tok_ids.at[pl.ds(base, _CH)], idx)
            pltpu.async_copy(tok_tab.at[idx], trows, sem).wait()
            pltpu.sync_copy(trows, tok_out.at[pl.ds(base, _CH)])
            pltpu.sync_copy(prev_ids.at[pl.ds(base, _CH)], idx)
            pltpu.async_copy(prev_tab.at[idx], prows, sem).wait()
            pltpu.sync_copy(prows, prev_out.at[pl.ds(base, _CH)])
            return 0

        lax.fori_loop(0, n_chunks, emb_step, 0)

        _fill_2d(prows, _CH, 128, 1.0)

        def deg_group(t, _):
            j0 = wid * e_per_w + t * 3 * _CH
            idescs = [
                pltpu.async_copy(dst_hbm.at[pl.ds(j0 + b * _CH, _CH)],
                                 didx[b], isem[b])
                for b in range(3)]
            sdescs = []
            for b in range(3):
                idescs[b].wait()
                sdescs.append(pltpu.async_copy(
                    prows, dacc.at[didx[b]], ssem[b], add=True))
            for b in range(3):
                sdescs[b].wait()
            return 0

        lax.fori_loop(0, e_chunks // 3, deg_group, 0)
        plsc.subcore_barrier()

        wdescs = [
            pltpu.async_copy(dacc.at[pl.ds(base_r + z * _CH, _CH)],
                             deg_out.at[c, pl.ds(base_r + z * _CH, _CH)],
                             sem)
            for z in range(n_slabs)]
        for d in wdescs:
            d.wait()

    return embed_kernel


_NB = 3


def _make_scatter_kernel(NACC, EPAD, H):
    e_per_w = EPAD // _NW
    e_chunks = e_per_w // _CH
    assert e_chunks % _NB == 0
    rows_per_tile = NACC // _NS
    n_slabs = rows_per_tile // _CH

    @functools.partial(
        pl.kernel,
        out_type=jax.ShapeDtypeStruct((_NC, NACC, H), jnp.float32),
        mesh=_sc_mesh(),
        scratch_types=[
            [pltpu.VMEM((_CH,), jnp.int32) for _ in range(_NB)],
            [pltpu.VMEM((_CH,), jnp.int32) for _ in range(_NB)],
            [pltpu.VMEM((_CH, H), jnp.float32) for _ in range(_NB)],
            pltpu.VMEM_SHARED((NACC, H), jnp.float32),
            [pltpu.SemaphoreType.DMA for _ in range(_NB)],
            [pltpu.SemaphoreType.DMA for _ in range(_NB)],
            [pltpu.SemaphoreType.DMA for _ in range(_NB)],
            [pltpu.SemaphoreType.DMA for _ in range(_NB)],
        ],
    )
    def scatter_kernel(g_hbm, src_hbm, dst_hbm, out_hbm,
                       sidx, didx, rows, acc, isem, idsem, gsem, ssem):
        c = lax.axis_index("c")
        s = lax.axis_index("s")
        wid = s * _NC + c
        e0 = wid * e_per_w

        _fill_2d(rows[0], _CH, H, 0.0)
        base_r = s * rows_per_tile
        zdescs = [
            pltpu.async_copy(rows[0], acc.at[pl.ds(base_r + z * _CH, _CH)],
                             gsem[0])
            for z in range(n_slabs)]
        for d in zdescs:
            d.wait()
        plsc.subcore_barrier()

        def group(t, _):
            j0 = e0 + t * _NB * _CH
            idescs = []
            for b in range(_NB):
                idescs.append(pltpu.async_copy(
                    src_hbm.at[pl.ds(j0 + b * _CH, _CH)], sidx[b], isem[b]))
                idescs.append(pltpu.async_copy(
                    dst_hbm.at[pl.ds(j0 + b * _CH, _CH)], didx[b], idsem[b]))
            descs = []
            for b in range(_NB):
                idescs[2 * b].wait()
                idescs[2 * b + 1].wait()
                descs.append(pltpu.async_copy(
                    g_hbm.at[sidx[b]], rows[b], gsem[b]))
            sdescs = []
            for b in range(_NB):
                descs[b].wait()
                sdescs.append(pltpu.async_copy(
                    rows[b], acc.at[didx[b]], ssem[b], add=True))
            for b in range(_NB):
                sdescs[b].wait()
            return 0

        lax.fori_loop(0, e_chunks // _NB, group, 0)
        plsc.subcore_barrier()

        wdescs = [
            pltpu.async_copy(acc.at[pl.ds(base_r + z * _CH, _CH)],
                             out_hbm.at[c, pl.ds(base_r + z * _CH, _CH)],
                             gsem[0])
            for z in range(n_slabs)]
        for d in wdescs:
            d.wait()

    return scatter_kernel


def _gelu(x):
    return 0.5 * x * (1.0 + lax.erf(x * 0.7071067811865476))


def _tc_input_body(it_ref, tok_ref, prev_ref, pos_ref, degp_ref, iter_ref,
                   wpos_ref, bpos_ref, wa_ref, wb_ref, wc_ref, wd_ref,
                   bin_ref, wenc_ref, g_ref, dinv_ref):
    it = it_ref[0]
    B = tok_ref.shape[0]
    p = jnp.dot(pos_ref[...], wpos_ref[...],
                preferred_element_type=jnp.float32) + bpos_ref[...]
    itrow = jnp.broadcast_to(iter_ref[pl.ds(it, 1), :], (B, iter_ref.shape[1]))
    h = (jnp.dot(tok_ref[...], wa_ref[...], preferred_element_type=jnp.float32)
         + jnp.dot(prev_ref[...], wb_ref[...], preferred_element_type=jnp.float32)
         + jnp.dot(p, wc_ref[...], preferred_element_type=jnp.float32)
         + jnp.dot(itrow, wd_ref[...], preferred_element_type=jnp.float32)
         + bin_ref[...])
    h = _gelu(h)
    deg = degp_ref[0, :, 0:1] + degp_ref[1, :, 0:1] + 1.0
    dinv = lax.rsqrt(deg)
    dinv_ref[...] = dinv
    g_ref[...] = jnp.dot(h, wenc_ref[...],
                         preferred_element_type=jnp.float32) * dinv


def _tc_mid_body(acc_ref, g_ref, dinv_ref, b_ref, w_ref, gout_ref):
    dinv = dinv_ref[...]
    t = _gelu(dinv * (acc_ref[0] + acc_ref[1] + g_ref[...]) + b_ref[...])
    gout_ref[...] = jnp.dot(t, w_ref[...],
                            preferred_element_type=jnp.float32) * dinv


def _tc_head_body(acc_ref, g_ref, dinv_ref, b_ref, wpred_ref, bpred_ref,
                  wconf_ref, bconf_ref, logits_ref, conf_ref):
    dinv = dinv_ref[...]
    h = _gelu(dinv * (acc_ref[0] + acc_ref[1] + g_ref[...]) + b_ref[...])
    logits_ref[...] = jnp.dot(h, wpred_ref[...],
                              preferred_element_type=jnp.float32) + bpred_ref[...]
    conf_ref[...] = jax.nn.sigmoid(
        jnp.dot(h, wconf_ref[...], preferred_element_type=jnp.float32)
        + bconf_ref[...])


def kernel(x, edge_index, iteration, tok_emb, prev_emb, iter_emb, W_pos,
           b_pos, W_in, b_in, W_enc0, b_enc0, W_bn0, b_bn0, W_bn1, b_bn1,
           W_dec0, b_dec0, W_pred, b_pred, W_conf, b_conf):
    N = x.shape[0]
    E = edge_index.shape[1]
    H = W_in.shape[1]
    DV = tok_emb.shape[1]
    DP = prev_emb.shape[1]
    V = W_pred.shape[1]

    NP = _round_up(N + 1, _CH * _NW)
    EPAD = _round_up(E, _CH * _NW * _NB)

    DPP = _round_up(DP, 128)

    tok_ids = jnp.pad(x[:, 0], (0, NP - N))
    prev_ids = jnp.pad(x[:, 1], (0, NP - N))
    pos = jnp.pad(x[:, 2:4].astype(jnp.float32), ((0, NP - N), (0, 0)))
    src_p = jnp.pad(edge_index[0], (0, EPAD - E), constant_values=N)
    dst_p = jnp.pad(edge_index[1], (0, EPAD - E), constant_values=N)
    it_arr = jnp.asarray([iteration], dtype=jnp.int32)
    prev_emb_p = jnp.pad(prev_emb, ((0, 0), (0, DPP - DP)))

    Wa = W_in[:DV]
    Wb = jnp.pad(W_in[DV:DV + DP], ((0, DPP - DP), (0, 0)))
    Wc = W_in[DV + DP:DV + 2 * DP]
    Wd = W_in[DV + 2 * DP:]

    tok_rows, prev_rows, degp = _make_embed_kernel(NP, EPAD, NP, DV, DPP)(
        tok_emb, prev_emb_p, tok_ids, prev_ids, dst_p)

    B = next(d for d in (1024, 896, 768, 640, 512, 448, 384, 256, 128)
             if NP % d == 0)
    grid = NP // B
    row_spec = lambda w: pl.BlockSpec((B, w), lambda i: (i, 0))
    full_spec = lambda a, b: pl.BlockSpec((a, b), lambda i: (0, 0))
    acc_spec = pl.BlockSpec((_NC, B, H), lambda i: (0, i, 0))
    vec_spec = lambda w: pl.BlockSpec((w,), lambda i: (0,))

    g1, dinv = pl.pallas_call(
        _tc_input_body,
        grid=(grid,),
        in_specs=[
            pl.BlockSpec(memory_space=pltpu.SMEM),
            row_spec(DV), row_spec(DPP), row_spec(2),
            pl.BlockSpec((_NC, B, 128), lambda i: (0, i, 0)),
            full_spec(*iter_emb.shape),
            full_spec(2, DP), vec_spec(DP),
            full_spec(DV, H), full_spec(DPP, H), full_spec(DP, H),
            full_spec(DP, H), vec_spec(H), full_spec(H, H),
        ],
        out_specs=[row_spec(H), row_spec(1)],
        out_shape=[
            jax.ShapeDtypeStruct((NP, H), jnp.float32),
            jax.ShapeDtypeStruct((NP, 1), jnp.float32),
        ],
    )(it_arr, tok_rows, prev_rows, pos, degp, iter_emb,
      W_pos, b_pos, Wa, Wb, Wc, Wd, b_in, W_enc0)

    scatter = _make_scatter_kernel(NP, EPAD, H)

    mid = pl.pallas_call(
        _tc_mid_body,
        grid=(grid,),
        in_specs=[acc_spec, row_spec(H), row_spec(1), vec_spec(H),
                  full_spec(H, H)],
        out_specs=row_spec(H),
        out_shape=jax.ShapeDtypeStruct((NP, H), jnp.float32),
    )

    g = g1
    for b_cur, W_next in ((b_enc0, W_bn0), (b_bn0, W_bn1), (b_bn1, W_dec0)):
        acc = scatter(g, src_p, dst_p)
        g = mid(acc, g, dinv, b_cur, W_next)

    acc = scatter(g, src_p, dst_p)
    logits, conf = pl.pallas_call(
        _tc_head_body,
        grid=(grid,),
        in_specs=[acc_spec, row_spec(H), row_spec(1), vec_spec(H),
                  full_spec(H, V), vec_spec(V), full_spec(H, 1), vec_spec(1)],
        out_specs=[row_spec(V), row_spec(1)],
        out_shape=[
            jax.ShapeDtypeStruct((NP, V), jnp.float32),
            jax.ShapeDtypeStruct((NP, 1), jnp.float32),
        ],
    )(acc, g, dinv, b_dec0, W_pred, b_pred, W_conf, b_conf)

    return (logits[:N], conf[:N])
